# baseline (speedup 1.0000x reference)
import jax
import jax.numpy as jnp
import numpy as np
from jax import lax
from jax.experimental import pallas as pl
from jax.experimental.pallas import tpu as pltpu
from jax.experimental.pallas import tpu_sc as plsc

_H = 64
_L = 64
_NNT = 8
_CARD = 32
_N = 1024

_NC = 2
_NS = 16
_NW = _NC * _NS
_NPW = _N // _NW
_STEPS = _NPW // 2
_T2_ROWS = 1056


def _pos_embed_np(seq_len, D=_H, max_len=10000.0):
    pos = np.arange(seq_len, dtype=np.float32)
    emb = np.zeros((seq_len, D), dtype=np.float32)
    for i in range(D // 2):
        omega_inv = max_len ** (2.0 * i / D)
        emb[:, 2 * i] = np.sin(pos / omega_inv)
        emb[:, 2 * i + 1] = np.cos(pos / omega_inv)
    return emb


_POS = _pos_embed_np(_L + 2)[:_L]

_BASE = np.empty((_L,), dtype=np.int32)
for _l in range(_L):
    _a, _m = _l // 16, _l % 16
    _BASE[_l] = (1024 + _a * 8 + _m) if _m < _NNT else (_a * 8 + (_m - _NNT)) * _CARD


def _body(t2_hbm, u_hbm, base_hbm, a_hbm, out0_hbm, out1_hbm,
          t2_v, u_v, base_v, a_v, bufs, wsems):
    cid = lax.axis_index("c")
    sid = lax.axis_index("s")
    n0 = cid * (_N // _NC) + sid * _NPW
    m0 = sid * _NPW

    pltpu.sync_copy(t2_hbm, t2_v)
    pltpu.sync_copy(u_hbm.at[pl.ds(n0 * 192, _NPW * 192)], u_v)
    pltpu.sync_copy(base_hbm, base_v)
    pltpu.sync_copy(a_hbm, a_v)

    lane = lax.iota(jnp.int32, 16)
    cat_lane = lane >= _NNT

    def splat(vec, j):
        return lax.gather(
            vec, jnp.full((16, 1), j, jnp.int32),
            lax.GatherDimensionNumbers(offset_dims=(), collapsed_slice_dims=(0,),
                                       start_index_map=(0,)),
            (1,), mode=lax.GatherScatterMode.PROMISE_IN_BOUNDS)

    a_regs = [[a_v[c, pl.ds(16 * kk, 16)] for kk in range(4)] for c in range(_NNT)]
    base_regs = [base_v[pl.ds(16 * k, 16)] for k in range(4)]

    def assemble(s, p):
        buf = bufs[p]
        for h in range(2):
            uoff = (2 * s + h) * 192
            for k in range(4):
                cls = plsc.load_gather(u_v, [uoff + 48 * k + 3 * lane])
                cls_i = (cls + 0.5).astype(jnp.int32)
                gidx = base_regs[k] + jnp.where(cat_lane, cls_i, 0)
                goff = gidx * _H
                for j0 in range(_NNT, 16, 2):
                    vals = []
                    for j in (j0, j0 + 1):
                        roff = splat(goff, j)
                        vals.append([plsc.load_gather(t2_v, [roff + 16 * kk + lane])
                                     for kk in range(4)])
                    for j, vv in zip((j0, j0 + 1), vals):
                        for kk in range(4):
                            buf[h, 16 * k + j, pl.ds(16 * kk, 16)] = vv[kk]
                for c0 in range(0, _NNT, 2):
                    vals = []
                    for c in (c0, c0 + 1):
                        l = 16 * k + c
                        voff = jnp.full((16,), uoff + 3 * l + 1, dtype=jnp.int32)
                        sc = plsc.load_gather(u_v, [voff])
                        toff = (1024 + k * _NNT + c) * _H
                        vals.append([t2_v[pl.ds(toff + 16 * kk, 16)]
                                     + sc * a_regs[c][kk] for kk in range(4)])
                    for c, vv in zip((c0, c0 + 1), vals):
                        for kk in range(4):
                            buf[h, 16 * k + c, pl.ds(16 * kk, 16)] = vv[kk]

    def start_wout(s, p):
        @pl.when(cid == 0)
        def _():
            pltpu.async_copy(bufs[p], out0_hbm.at[pl.ds(m0 + 2 * s, 2)], wsems[p])

        @pl.when(cid == 1)
        def _():
            pltpu.async_copy(bufs[p], out1_hbm.at[pl.ds(m0 + 2 * s, 2)], wsems[p])

    def wait_wout(s, p):
        @pl.when(cid == 0)
        def _():
            pltpu.make_async_copy(bufs[p], out0_hbm.at[pl.ds(m0 + 2 * s, 2)],
                                  wsems[p]).wait()

        @pl.when(cid == 1)
        def _():
            pltpu.make_async_copy(bufs[p], out1_hbm.at[pl.ds(m0 + 2 * s, 2)],
                                  wsems[p]).wait()

    def step(s, p):
        @pl.when(s >= 2)
        def _():
            wait_wout(s - 2, p)
        assemble(s, p)
        start_wout(s, p)

    def loop_body(i, carry):
        step(2 * i, 0)
        step(2 * i + 1, 1)
        return carry

    lax.fori_loop(0, _STEPS // 2, loop_body, 0)

    wait_wout(_STEPS - 2, 0)
    wait_wout(_STEPS - 1, 1)


def kernel(u_in, w_num, cat_table):
    pos = jnp.asarray(_POS)
    pos_cat = pos.reshape(4, 16, _H)[:, _NNT:, :]
    pos_num = pos.reshape(4, 16, _H)[:, :_NNT, :]
    w8 = w_num[: _NNT * _H].reshape(_NNT, _H)
    t_cat = (cat_table.reshape(_NNT, _CARD, _H)[None] +
             pos_cat[:, :, None, :]).reshape(1024, _H)
    t_num = (pos_num - w8[None]).reshape(32, _H)
    t2 = jnp.concatenate([t_cat, t_num], axis=0).reshape(-1)
    a8 = 2.0 * w8
    u_flat = u_in.reshape(-1)
    base = jnp.asarray(_BASE)

    mesh = plsc.VectorSubcoreMesh(core_axis_name="c", subcore_axis_name="s",
                                  num_cores=_NC, num_subcores=_NS)
    run = pl.kernel(
        _body,
        out_type=(jax.ShapeDtypeStruct((_N // _NC, _L, _H), jnp.float32),
                  jax.ShapeDtypeStruct((_N // _NC, _L, _H), jnp.float32)),
        mesh=mesh,
        scratch_types=[
            pltpu.VMEM((_T2_ROWS * _H,), jnp.float32),
            pltpu.VMEM((_NPW * 192,), jnp.float32),
            pltpu.VMEM((_L,), jnp.int32),
            pltpu.VMEM((_NNT, _H), jnp.float32),
            [pltpu.VMEM((2, _L, _H), jnp.float32)] * 2,
            [pltpu.SemaphoreType.DMA] * 2,
        ],
        compiler_params=pltpu.CompilerParams(needs_layout_passes=False,
                                             use_tc_tiling_on_sc=True),
    )
    o0, o1 = run(t2, u_flat, base, a8)
    return jnp.concatenate([o0, o1], axis=0)


# speedup vs baseline: 1.1940x; 1.1940x over previous
import jax
import jax.numpy as jnp
import numpy as np
from jax import lax
from jax.experimental import pallas as pl
from jax.experimental.pallas import tpu as pltpu
from jax.experimental.pallas import tpu_sc as plsc

_H = 64
_L = 64
_NNT = 8
_CARD = 32
_N = 1024

_NC = 2
_NS = 16
_NW = _NC * _NS
_NPW = _N // _NW
_STEPS = _NPW // 2
_T2_ROWS = 1056


def _pos_embed_np(seq_len, D=_H, max_len=10000.0):
    pos = np.arange(seq_len, dtype=np.float32)
    emb = np.zeros((seq_len, D), dtype=np.float32)
    for i in range(D // 2):
        omega_inv = max_len ** (2.0 * i / D)
        emb[:, 2 * i] = np.sin(pos / omega_inv)
        emb[:, 2 * i + 1] = np.cos(pos / omega_inv)
    return emb


_POS = _pos_embed_np(_L + 2)[:_L]

_BASE = np.empty((_L,), dtype=np.int32)
for _l in range(_L):
    _a, _m = _l // 16, _l % 16
    _BASE[_l] = (1024 + _a * 8 + _m) if _m < _NNT else (_a * 8 + (_m - _NNT)) * _CARD


def _body(t2_hbm, u_hbm, base_hbm, a_hbm, out_hbm,
          t2_v, u_v, base_v, a_v, bufs, wsems):
    wid = lax.axis_index("s") * _NC + lax.axis_index("c")
    n0 = wid * _NPW

    pltpu.sync_copy(t2_hbm, t2_v)
    pltpu.sync_copy(u_hbm.at[pl.ds(n0 * 192, _NPW * 192)], u_v)
    pltpu.sync_copy(base_hbm, base_v)
    pltpu.sync_copy(a_hbm, a_v)

    lane = lax.iota(jnp.int32, 16)
    cat_lane = lane >= _NNT

    def splat(vec, j):
        return lax.gather(
            vec, jnp.full((16, 1), j, jnp.int32),
            lax.GatherDimensionNumbers(offset_dims=(), collapsed_slice_dims=(0,),
                                       start_index_map=(0,)),
            (1,), mode=lax.GatherScatterMode.PROMISE_IN_BOUNDS)

    a_regs = [[a_v[c, pl.ds(16 * kk, 16)] for kk in range(4)] for c in range(_NNT)]
    base_regs = [base_v[pl.ds(16 * k, 16)] for k in range(4)]

    def assemble(s, p):
        buf = bufs[p]
        for h in range(2):
            uoff = (2 * s + h) * 192
            for k in range(4):
                cls = plsc.load_gather(u_v, [uoff + 48 * k + 3 * lane])
                vals16 = plsc.load_gather(u_v, [uoff + 48 * k + 3 * lane + 1])
                cls_i = (cls + 0.5).astype(jnp.int32)
                gidx = base_regs[k] + jnp.where(cat_lane, cls_i, 0)
                goff = gidx * _H
                for j0 in range(_NNT, 16, 2):
                    vals = []
                    for j in (j0, j0 + 1):
                        roff = splat(goff, j)
                        vals.append([plsc.load_gather(t2_v, [roff + 16 * kk + lane])
                                     for kk in range(4)])
                    for j, vv in zip((j0, j0 + 1), vals):
                        for kk in range(4):
                            buf[h, 16 * k + j, pl.ds(16 * kk, 16)] = vv[kk]
                for c0 in range(0, _NNT, 2):
                    vals = []
                    for c in (c0, c0 + 1):
                        sc = splat(vals16, c)
                        toff = (1024 + k * _NNT + c) * _H
                        vals.append([t2_v[pl.ds(toff + 16 * kk, 16)]
                                     + sc * a_regs[c][kk] for kk in range(4)])
                    for c, vv in zip((c0, c0 + 1), vals):
                        for kk in range(4):
                            buf[h, 16 * k + c, pl.ds(16 * kk, 16)] = vv[kk]

    def start_wout(s, p):
        pltpu.async_copy(bufs[p], out_hbm.at[pl.ds(n0 + 2 * s, 2)], wsems[p])

    def wait_wout(s, p):
        pltpu.make_async_copy(bufs[p], out_hbm.at[pl.ds(n0 + 2 * s, 2)],
                              wsems[p]).wait()

    def step(s, p):
        @pl.when(s >= 2)
        def _():
            wait_wout(s - 2, p)
        assemble(s, p)
        start_wout(s, p)

    def loop_body(i, carry):
        step(2 * i, 0)
        step(2 * i + 1, 1)
        return carry

    lax.fori_loop(0, _STEPS // 2, loop_body, 0)

    wait_wout(_STEPS - 2, 0)
    wait_wout(_STEPS - 1, 1)


def kernel(u_in, w_num, cat_table):
    pos = jnp.asarray(_POS)
    pos_cat = pos.reshape(4, 16, _H)[:, _NNT:, :]
    pos_num = pos.reshape(4, 16, _H)[:, :_NNT, :]
    w8 = w_num[: _NNT * _H].reshape(_NNT, _H)
    t_cat = (cat_table.reshape(_NNT, _CARD, _H)[None] +
             pos_cat[:, :, None, :]).reshape(1024, _H)
    t_num = (pos_num - w8[None]).reshape(32, _H)
    t2 = jnp.concatenate([t_cat, t_num], axis=0).reshape(-1)
    a8 = 2.0 * w8
    u_flat = u_in.reshape(-1)
    base = jnp.asarray(_BASE)

    mesh = plsc.VectorSubcoreMesh(core_axis_name="c", subcore_axis_name="s",
                                  num_cores=_NC, num_subcores=_NS)
    run = pl.kernel(
        _body,
        out_type=jax.ShapeDtypeStruct((_N, _L, _H), jnp.float32),
        mesh=mesh,
        scratch_types=[
            pltpu.VMEM((_T2_ROWS * _H,), jnp.float32),
            pltpu.VMEM((_NPW * 192,), jnp.float32),
            pltpu.VMEM((_L,), jnp.int32),
            pltpu.VMEM((_NNT, _H), jnp.float32),
            [pltpu.VMEM((2, _L, _H), jnp.float32)] * 2,
            [pltpu.SemaphoreType.DMA] * 2,
        ],
        compiler_params=pltpu.CompilerParams(needs_layout_passes=False,
                                             use_tc_tiling_on_sc=True),
    )
    return run(t2, u_flat, base, a8)


# speedup vs baseline: 1.1982x; 1.0035x over previous
import jax
import jax.numpy as jnp
import numpy as np
from jax import lax
from jax.experimental import pallas as pl
from jax.experimental.pallas import tpu as pltpu
from jax.experimental.pallas import tpu_sc as plsc

_H = 64
_L = 64
_NNT = 8
_CARD = 32
_N = 1024

_NC = 2
_NS = 16
_NW = _NC * _NS
_NPW = _N // _NW
_STEPS = _NPW // 2
_T2_ROWS = 1056


def _pos_embed_np(seq_len, D=_H, max_len=10000.0):
    pos = np.arange(seq_len, dtype=np.float32)
    emb = np.zeros((seq_len, D), dtype=np.float32)
    for i in range(D // 2):
        omega_inv = max_len ** (2.0 * i / D)
        emb[:, 2 * i] = np.sin(pos / omega_inv)
        emb[:, 2 * i + 1] = np.cos(pos / omega_inv)
    return emb


_POS = _pos_embed_np(_L + 2)[:_L]

_BASE = np.empty((_L,), dtype=np.int32)
for _l in range(_L):
    _a, _m = _l // 16, _l % 16
    _BASE[_l] = (1024 + _a * 8 + _m) if _m < _NNT else (_a * 8 + (_m - _NNT)) * _CARD


def _body(t2_hbm, u_hbm, base_hbm, a_hbm, out_hbm,
          t2_v, u_v, base_v, a_v, bufs, wsems):
    wid = lax.axis_index("s") * _NC + lax.axis_index("c")
    n0 = wid * _NPW

    pltpu.sync_copy(t2_hbm, t2_v)
    pltpu.sync_copy(u_hbm.at[pl.ds(n0 * 192, _NPW * 192)], u_v)
    pltpu.sync_copy(base_hbm, base_v)
    pltpu.sync_copy(a_hbm, a_v)

    lane = lax.iota(jnp.int32, 16)
    cat_lane = lane >= _NNT

    def splat(vec, j):
        return lax.gather(
            vec, jnp.full((16, 1), j, jnp.int32),
            lax.GatherDimensionNumbers(offset_dims=(), collapsed_slice_dims=(0,),
                                       start_index_map=(0,)),
            (1,), mode=lax.GatherScatterMode.PROMISE_IN_BOUNDS)

    a_regs = [[a_v[c, pl.ds(16 * kk, 16)] for kk in range(4)] for c in range(_NNT)]
    base_regs = [base_v[pl.ds(16 * k, 16)] for k in range(4)]

    def assemble(s, p):
        buf = bufs[p]
        for h in range(2):
            uoff = (2 * s + h) * 192
            for k in range(4):
                cls = plsc.load_gather(u_v, [uoff + 48 * k + 3 * lane])
                vals16 = plsc.load_gather(u_v, [uoff + 48 * k + 3 * lane + 1])
                cls_i = (cls + 0.5).astype(jnp.int32)
                gidx = base_regs[k] + jnp.where(cat_lane, cls_i, 0)
                goff = gidx * _H
                for j0 in range(_NNT, 16, 4):
                    rows = range(j0, j0 + 4)
                    vals = []
                    for j in rows:
                        roff = splat(goff, j)
                        vals.append([plsc.load_gather(t2_v, [roff + 16 * kk + lane])
                                     for kk in range(4)])
                    for j, vv in zip(rows, vals):
                        for kk in range(4):
                            buf[h, 16 * k + j, pl.ds(16 * kk, 16)] = vv[kk]
                for c0 in range(0, _NNT, 4):
                    rows = range(c0, c0 + 4)
                    vals = []
                    for c in rows:
                        sc = splat(vals16, c)
                        toff = (1024 + k * _NNT + c) * _H
                        vals.append([t2_v[pl.ds(toff + 16 * kk, 16)]
                                     + sc * a_regs[c][kk] for kk in range(4)])
                    for c, vv in zip(rows, vals):
                        for kk in range(4):
                            buf[h, 16 * k + c, pl.ds(16 * kk, 16)] = vv[kk]

    def start_wout(s, p):
        pltpu.async_copy(bufs[p], out_hbm.at[pl.ds(n0 + 2 * s, 2)], wsems[p])

    def wait_wout(s, p):
        pltpu.make_async_copy(bufs[p], out_hbm.at[pl.ds(n0 + 2 * s, 2)],
                              wsems[p]).wait()

    def step(s, p):
        @pl.when(s >= 2)
        def _():
            wait_wout(s - 2, p)
        assemble(s, p)
        start_wout(s, p)

    def loop_body(i, carry):
        step(2 * i, 0)
        step(2 * i + 1, 1)
        return carry

    lax.fori_loop(0, _STEPS // 2, loop_body, 0)

    wait_wout(_STEPS - 2, 0)
    wait_wout(_STEPS - 1, 1)


def kernel(u_in, w_num, cat_table):
    pos = jnp.asarray(_POS)
    pos_cat = pos.reshape(4, 16, _H)[:, _NNT:, :]
    pos_num = pos.reshape(4, 16, _H)[:, :_NNT, :]
    w8 = w_num[: _NNT * _H].reshape(_NNT, _H)
    t_cat = (cat_table.reshape(_NNT, _CARD, _H)[None] +
             pos_cat[:, :, None, :]).reshape(1024, _H)
    t_num = (pos_num - w8[None]).reshape(32, _H)
    t2 = jnp.concatenate([t_cat, t_num], axis=0).reshape(-1)
    a8 = 2.0 * w8
    u_flat = u_in.reshape(-1)
    base = jnp.asarray(_BASE)

    mesh = plsc.VectorSubcoreMesh(core_axis_name="c", subcore_axis_name="s",
                                  num_cores=_NC, num_subcores=_NS)
    run = pl.kernel(
        _body,
        out_type=jax.ShapeDtypeStruct((_N, _L, _H), jnp.float32),
        mesh=mesh,
        scratch_types=[
            pltpu.VMEM((_T2_ROWS * _H,), jnp.float32),
            pltpu.VMEM((_NPW * 192,), jnp.float32),
            pltpu.VMEM((_L,), jnp.int32),
            pltpu.VMEM((_NNT, _H), jnp.float32),
            [pltpu.VMEM((2, _L, _H), jnp.float32)] * 2,
            [pltpu.SemaphoreType.DMA] * 2,
        ],
        compiler_params=pltpu.CompilerParams(needs_layout_passes=False,
                                             use_tc_tiling_on_sc=True),
    )
    return run(t2, u_flat, base, a8)
